# Initial kernel scaffold; baseline (speedup 1.0000x reference)
import jax
import jax.numpy as jnp
from jax.experimental import pallas as pl


def kernel(x, W_enc, b_enc, emb, W_dec, b_dec):
    raise NotImplementedError("write your pallas kernel here")


# speedup vs baseline: 1.0856x; 1.0856x over previous
import functools

import jax
import jax.numpy as jnp
from jax import lax
from jax.experimental import pallas as pl
from jax.experimental.pallas import tpu as pltpu
from jax.experimental.pallas import tpu_sc as plsc

N_TOK = 18432
INPUT_DIM = 768
LATENT_DIM = 256
NUM_EMB = 8192

BT = 128
BE = 1024
CHUNK_E = 2048

SC_NC = 2
SC_NS = 16
SC_NW = SC_NC * SC_NS
ROWS_PER_W = N_TOK // SC_NW
CHUNK = 96
N_CHUNKS = ROWS_PER_W // CHUNK


def _embdec_body(emb_ref, wdt_ref, bd_ref, dec_ref, esq_ref):
    e = emb_ref[...]
    dec_ref[...] = (
        lax.dot_general(e, wdt_ref[...], (((1,), (0,)), ((), ())),
                        preferred_element_type=jnp.float32)
        + bd_ref[...]
    )
    esq_ref[...] = jnp.sum(e * e, axis=1, keepdims=True)


def _distance_body(x_ref, wet_ref, be_ref, embt_ref, esq_ref, z_ref, idx_ref):
    z = (
        lax.dot_general(x_ref[...], wet_ref[...], (((1,), (0,)), ((), ())),
                        preferred_element_type=jnp.float32)
        + be_ref[...]
    )
    z_ref[...] = z
    zsq = jnp.sum(z * z, axis=1, keepdims=True)
    s = lax.dot_general(z, embt_ref[...], (((1,), (0,)), ((), ())),
                        preferred_element_type=jnp.float32)
    d = (zsq + esq_ref[...]) - 2.0 * s
    ii = lax.broadcasted_iota(jnp.int32, (BT, CHUNK_E), 1)
    M = None
    I = None
    for c in range(NUM_EMB // CHUNK_E):
        dc = d[:, c * CHUNK_E:(c + 1) * CHUNK_E]
        m_c = jnp.min(dc, axis=1, keepdims=True)
        i_c = jnp.min(jnp.where(dc == m_c, ii, CHUNK_E), axis=1,
                      keepdims=True) + c * CHUNK_E
        if c == 0:
            M = m_c.astype(jnp.bfloat16).astype(jnp.float32)
            I = i_c
        else:
            keep = M <= m_c
            I = jnp.where(keep, I, i_c)
            M = jnp.where(keep, M, m_c).astype(jnp.bfloat16).astype(
                jnp.float32)
    idx_ref[...] = I


def _gather_body(emb_hbm, dec_hbm, idx_hbm, zq_out, xr_out,
                 idx_v, ebuf, dbuf, sem_e, sem_d):
    wid = lax.axis_index("s") * SC_NC + lax.axis_index("c")
    base = wid * ROWS_PER_W
    for j in range(N_CHUNKS):
        off = base + j * CHUNK
        pltpu.sync_copy(idx_hbm.at[pl.ds(off, CHUNK)], idx_v)
        cp_e = pltpu.async_copy(emb_hbm.at[idx_v], ebuf, sem_e)
        cp_d = pltpu.async_copy(dec_hbm.at[idx_v], dbuf, sem_d)
        cp_e.wait()
        cp_d.wait()
        pltpu.sync_copy(ebuf, zq_out.at[pl.ds(off, CHUNK)])
        pltpu.sync_copy(dbuf, xr_out.at[pl.ds(off, CHUNK)])


@functools.cache
def _sc_gather():
    return pl.kernel(
        _gather_body,
        out_type=[
            jax.ShapeDtypeStruct((N_TOK, LATENT_DIM), jnp.float32),
            jax.ShapeDtypeStruct((N_TOK, INPUT_DIM), jnp.float32),
        ],
        mesh=plsc.VectorSubcoreMesh(core_axis_name="c", subcore_axis_name="s"),
        scratch_types=[
            pltpu.VMEM((CHUNK,), jnp.int32),
            pltpu.VMEM((CHUNK, LATENT_DIM), jnp.float32),
            pltpu.VMEM((CHUNK, INPUT_DIM), jnp.float32),
            pltpu.SemaphoreType.DMA,
            pltpu.SemaphoreType.DMA,
        ],
    )


def kernel(x, W_enc, b_enc, emb, W_dec, b_dec):
    wet = W_enc.T
    wdt = W_dec.T
    embt = emb.T
    be2 = b_enc.reshape(1, LATENT_DIM)
    bd2 = b_dec.reshape(1, INPUT_DIM)

    dec, esq_col = pl.pallas_call(
        _embdec_body,
        grid=(NUM_EMB // BE,),
        in_specs=[
            pl.BlockSpec((BE, LATENT_DIM), lambda i: (i, 0)),
            pl.BlockSpec((LATENT_DIM, INPUT_DIM), lambda i: (0, 0)),
            pl.BlockSpec((1, INPUT_DIM), lambda i: (0, 0)),
        ],
        out_specs=[
            pl.BlockSpec((BE, INPUT_DIM), lambda i: (i, 0)),
            pl.BlockSpec((BE, 1), lambda i: (i, 0)),
        ],
        out_shape=[
            jax.ShapeDtypeStruct((NUM_EMB, INPUT_DIM), jnp.float32),
            jax.ShapeDtypeStruct((NUM_EMB, 1), jnp.float32),
        ],
    )(emb, wdt, bd2)

    esq_row = esq_col.reshape(1, NUM_EMB)

    z, idx2d = pl.pallas_call(
        _distance_body,
        grid=(N_TOK // BT,),
        in_specs=[
            pl.BlockSpec((BT, INPUT_DIM), lambda i: (i, 0)),
            pl.BlockSpec((INPUT_DIM, LATENT_DIM), lambda i: (0, 0)),
            pl.BlockSpec((1, LATENT_DIM), lambda i: (0, 0)),
            pl.BlockSpec((LATENT_DIM, NUM_EMB), lambda i: (0, 0)),
            pl.BlockSpec((1, NUM_EMB), lambda i: (0, 0)),
        ],
        out_specs=[
            pl.BlockSpec((BT, LATENT_DIM), lambda i: (i, 0)),
            pl.BlockSpec((BT, 1), lambda i: (i, 0)),
        ],
        out_shape=[
            jax.ShapeDtypeStruct((N_TOK, LATENT_DIM), jnp.float32),
            jax.ShapeDtypeStruct((N_TOK, 1), jnp.int32),
        ],
    )(x, wet, be2, embt, esq_row)

    idx = idx2d.reshape(N_TOK)
    z_q, x_recon = _sc_gather()(emb, dec, idx)
    return (x_recon, z, z_q, idx)


# speedup vs baseline: 1.3512x; 1.2447x over previous
import functools

import jax
import jax.numpy as jnp
from jax import lax
from jax.experimental import pallas as pl
from jax.experimental.pallas import tpu as pltpu
from jax.experimental.pallas import tpu_sc as plsc

N_TOK = 18432
INPUT_DIM = 768
LATENT_DIM = 256
NUM_EMB = 8192

BT = 512
BE = 1024
CHUNK_E = 2048

SC_NC = 2
SC_NS = 16
SC_NW = SC_NC * SC_NS
ROWS_PER_W = N_TOK // SC_NW
CHUNK = 96
N_CHUNKS = ROWS_PER_W // CHUNK


def _embdec_body(emb_ref, wdt_ref, bd_ref, dec_ref, esq_ref):
    e = emb_ref[...]
    dec_ref[...] = (
        lax.dot_general(e, wdt_ref[...], (((1,), (0,)), ((), ())),
                        preferred_element_type=jnp.float32)
        + bd_ref[...]
    )
    esq_ref[...] = jnp.sum(e * e, axis=1, keepdims=True)


def _distance_body(x_ref, wet_ref, be_ref, embt_ref, esq_ref, z_ref, idx_ref):
    z = (
        lax.dot_general(x_ref[...], wet_ref[...], (((1,), (0,)), ((), ())),
                        preferred_element_type=jnp.float32)
        + be_ref[...]
    )
    z_ref[...] = z
    zsq = jnp.sum(z * z, axis=1, keepdims=True)
    ii = lax.broadcasted_iota(jnp.int32, (BT, CHUNK_E), 1)
    M = None
    I = None
    for c in range(NUM_EMB // CHUNK_E):
        sc = lax.dot_general(
            z, embt_ref[:, c * CHUNK_E:(c + 1) * CHUNK_E],
            (((1,), (0,)), ((), ())), preferred_element_type=jnp.float32)
        dc = (zsq + esq_ref[:, c * CHUNK_E:(c + 1) * CHUNK_E]) - 2.0 * sc
        m_c = jnp.min(dc, axis=1, keepdims=True)
        i_c = jnp.min(jnp.where(dc == m_c, ii, CHUNK_E), axis=1,
                      keepdims=True) + c * CHUNK_E
        if c == 0:
            M = m_c.astype(jnp.bfloat16).astype(jnp.float32)
            I = i_c
        else:
            keep = M <= m_c
            I = jnp.where(keep, I, i_c)
            M = jnp.where(keep, M, m_c).astype(jnp.bfloat16).astype(
                jnp.float32)
    idx_ref[...] = I


def _gather_body(emb_hbm, dec_hbm, idx_hbm, zq_out, xr_out,
                 idx_v, ebuf, dbuf, sem_e, sem_d):
    wid = lax.axis_index("s") * SC_NC + lax.axis_index("c")
    base = wid * ROWS_PER_W
    for j in range(N_CHUNKS):
        off = base + j * CHUNK
        pltpu.sync_copy(idx_hbm.at[pl.ds(off, CHUNK)], idx_v)
        cp_e = pltpu.async_copy(emb_hbm.at[idx_v], ebuf, sem_e)
        cp_d = pltpu.async_copy(dec_hbm.at[idx_v], dbuf, sem_d)
        cp_e.wait()
        cp_d.wait()
        pltpu.sync_copy(ebuf, zq_out.at[pl.ds(off, CHUNK)])
        pltpu.sync_copy(dbuf, xr_out.at[pl.ds(off, CHUNK)])


@functools.cache
def _sc_gather():
    return pl.kernel(
        _gather_body,
        out_type=[
            jax.ShapeDtypeStruct((N_TOK, LATENT_DIM), jnp.float32),
            jax.ShapeDtypeStruct((N_TOK, INPUT_DIM), jnp.float32),
        ],
        mesh=plsc.VectorSubcoreMesh(core_axis_name="c", subcore_axis_name="s"),
        scratch_types=[
            pltpu.VMEM((CHUNK,), jnp.int32),
            pltpu.VMEM((CHUNK, LATENT_DIM), jnp.float32),
            pltpu.VMEM((CHUNK, INPUT_DIM), jnp.float32),
            pltpu.SemaphoreType.DMA,
            pltpu.SemaphoreType.DMA,
        ],
    )


def kernel(x, W_enc, b_enc, emb, W_dec, b_dec):
    wet = W_enc.T
    wdt = W_dec.T
    embt = emb.T
    be2 = b_enc.reshape(1, LATENT_DIM)
    bd2 = b_dec.reshape(1, INPUT_DIM)

    dec, esq_col = pl.pallas_call(
        _embdec_body,
        grid=(NUM_EMB // BE,),
        in_specs=[
            pl.BlockSpec((BE, LATENT_DIM), lambda i: (i, 0)),
            pl.BlockSpec((LATENT_DIM, INPUT_DIM), lambda i: (0, 0)),
            pl.BlockSpec((1, INPUT_DIM), lambda i: (0, 0)),
        ],
        out_specs=[
            pl.BlockSpec((BE, INPUT_DIM), lambda i: (i, 0)),
            pl.BlockSpec((BE, 1), lambda i: (i, 0)),
        ],
        out_shape=[
            jax.ShapeDtypeStruct((NUM_EMB, INPUT_DIM), jnp.float32),
            jax.ShapeDtypeStruct((NUM_EMB, 1), jnp.float32),
        ],
    )(emb, wdt, bd2)

    esq_row = esq_col.reshape(1, NUM_EMB)

    z, idx2d = pl.pallas_call(
        _distance_body,
        grid=(N_TOK // BT,),
        in_specs=[
            pl.BlockSpec((BT, INPUT_DIM), lambda i: (i, 0)),
            pl.BlockSpec((INPUT_DIM, LATENT_DIM), lambda i: (0, 0)),
            pl.BlockSpec((1, LATENT_DIM), lambda i: (0, 0)),
            pl.BlockSpec((LATENT_DIM, NUM_EMB), lambda i: (0, 0)),
            pl.BlockSpec((1, NUM_EMB), lambda i: (0, 0)),
        ],
        out_specs=[
            pl.BlockSpec((BT, LATENT_DIM), lambda i: (i, 0)),
            pl.BlockSpec((BT, 1), lambda i: (i, 0)),
        ],
        out_shape=[
            jax.ShapeDtypeStruct((N_TOK, LATENT_DIM), jnp.float32),
            jax.ShapeDtypeStruct((N_TOK, 1), jnp.int32),
        ],
    )(x, wet, be2, embt, esq_row)

    idx = idx2d.reshape(N_TOK)
    z_q, x_recon = _sc_gather()(emb, dec, idx)
    return (x_recon, z, z_q, idx)


# speedup vs baseline: 1.3753x; 1.0178x over previous
import functools

import jax
import jax.numpy as jnp
from jax import lax
from jax.experimental import pallas as pl
from jax.experimental.pallas import tpu as pltpu
from jax.experimental.pallas import tpu_sc as plsc

N_TOK = 18432
INPUT_DIM = 768
LATENT_DIM = 256
NUM_EMB = 8192

BT = 512
BE = 1024
CHUNK_E = 2048

SC_NC = 2
SC_NS = 16
SC_NW = SC_NC * SC_NS
ROWS_PER_W = N_TOK // SC_NW
CHUNK = 96
N_CHUNKS = ROWS_PER_W // CHUNK


def _embdec_body(emb_ref, wdt_ref, bd_ref, dec_ref, embt_ref, esq_ref):
    e = emb_ref[...]
    dec_ref[...] = (
        lax.dot_general(e, wdt_ref[...], (((1,), (0,)), ((), ())),
                        preferred_element_type=jnp.float32)
        + bd_ref[...]
    )
    et = e.T
    embt_ref[...] = et
    esq_ref[...] = jnp.sum(et * et, axis=0, keepdims=True)


def _distance_body(x_ref, wet_ref, be_ref, embt_ref, esq_ref, z_ref, idx_ref):
    z = (
        lax.dot_general(x_ref[...], wet_ref[...], (((1,), (0,)), ((), ())),
                        preferred_element_type=jnp.float32)
        + be_ref[...]
    )
    z_ref[...] = z
    zsq = jnp.sum(z * z, axis=1, keepdims=True)
    ii = lax.broadcasted_iota(jnp.int32, (BT, CHUNK_E), 1)
    M = None
    I = None
    for c in range(NUM_EMB // CHUNK_E):
        sc = lax.dot_general(
            z, embt_ref[:, c * CHUNK_E:(c + 1) * CHUNK_E],
            (((1,), (0,)), ((), ())), preferred_element_type=jnp.float32)
        dc = (zsq + esq_ref[:, c * CHUNK_E:(c + 1) * CHUNK_E]) - 2.0 * sc
        m_c = jnp.min(dc, axis=1, keepdims=True)
        i_c = jnp.min(jnp.where(dc == m_c, ii, CHUNK_E), axis=1,
                      keepdims=True) + c * CHUNK_E
        if c == 0:
            M = m_c.astype(jnp.bfloat16).astype(jnp.float32)
            I = i_c
        else:
            keep = M <= m_c
            I = jnp.where(keep, I, i_c)
            M = jnp.where(keep, M, m_c).astype(jnp.bfloat16).astype(
                jnp.float32)
    idx_ref[...] = I


def _gather_body(emb_hbm, dec_hbm, idx_hbm, zq_out, xr_out,
                 idx_v, ebuf, dbuf, sem_e, sem_d):
    wid = lax.axis_index("s") * SC_NC + lax.axis_index("c")
    base = wid * ROWS_PER_W
    for j in range(N_CHUNKS):
        off = base + j * CHUNK
        pltpu.sync_copy(idx_hbm.at[pl.ds(off, CHUNK)], idx_v)
        cp_e = pltpu.async_copy(emb_hbm.at[idx_v], ebuf, sem_e)
        cp_d = pltpu.async_copy(dec_hbm.at[idx_v], dbuf, sem_d)
        cp_e.wait()
        cp_d.wait()
        pltpu.sync_copy(ebuf, zq_out.at[pl.ds(off, CHUNK)])
        pltpu.sync_copy(dbuf, xr_out.at[pl.ds(off, CHUNK)])


@functools.cache
def _sc_gather():
    return pl.kernel(
        _gather_body,
        out_type=[
            jax.ShapeDtypeStruct((N_TOK, LATENT_DIM), jnp.float32),
            jax.ShapeDtypeStruct((N_TOK, INPUT_DIM), jnp.float32),
        ],
        mesh=plsc.VectorSubcoreMesh(core_axis_name="c", subcore_axis_name="s"),
        scratch_types=[
            pltpu.VMEM((CHUNK,), jnp.int32),
            pltpu.VMEM((CHUNK, LATENT_DIM), jnp.float32),
            pltpu.VMEM((CHUNK, INPUT_DIM), jnp.float32),
            pltpu.SemaphoreType.DMA,
            pltpu.SemaphoreType.DMA,
        ],
    )


def kernel(x, W_enc, b_enc, emb, W_dec, b_dec):
    wet = W_enc.T
    wdt = W_dec.T
    be2 = b_enc.reshape(1, LATENT_DIM)
    bd2 = b_dec.reshape(1, INPUT_DIM)

    dec, embt, esq_row = pl.pallas_call(
        _embdec_body,
        grid=(NUM_EMB // BE,),
        in_specs=[
            pl.BlockSpec((BE, LATENT_DIM), lambda i: (i, 0)),
            pl.BlockSpec((LATENT_DIM, INPUT_DIM), lambda i: (0, 0)),
            pl.BlockSpec((1, INPUT_DIM), lambda i: (0, 0)),
        ],
        out_specs=[
            pl.BlockSpec((BE, INPUT_DIM), lambda i: (i, 0)),
            pl.BlockSpec((LATENT_DIM, BE), lambda i: (0, i)),
            pl.BlockSpec((1, BE), lambda i: (0, i)),
        ],
        out_shape=[
            jax.ShapeDtypeStruct((NUM_EMB, INPUT_DIM), jnp.float32),
            jax.ShapeDtypeStruct((LATENT_DIM, NUM_EMB), jnp.float32),
            jax.ShapeDtypeStruct((1, NUM_EMB), jnp.float32),
        ],
    )(emb, wdt, bd2)

    z, idx2d = pl.pallas_call(
        _distance_body,
        grid=(N_TOK // BT,),
        in_specs=[
            pl.BlockSpec((BT, INPUT_DIM), lambda i: (i, 0)),
            pl.BlockSpec((INPUT_DIM, LATENT_DIM), lambda i: (0, 0)),
            pl.BlockSpec((1, LATENT_DIM), lambda i: (0, 0)),
            pl.BlockSpec((LATENT_DIM, NUM_EMB), lambda i: (0, 0)),
            pl.BlockSpec((1, NUM_EMB), lambda i: (0, 0)),
        ],
        out_specs=[
            pl.BlockSpec((BT, LATENT_DIM), lambda i: (i, 0)),
            pl.BlockSpec((BT, 1), lambda i: (i, 0)),
        ],
        out_shape=[
            jax.ShapeDtypeStruct((N_TOK, LATENT_DIM), jnp.float32),
            jax.ShapeDtypeStruct((N_TOK, 1), jnp.int32),
        ],
    )(x, wet, be2, embt, esq_row)

    idx = idx2d.reshape(N_TOK)
    z_q, x_recon = _sc_gather()(emb, dec, idx)
    return (x_recon, z, z_q, idx)
